# baseline (speedup 1.0000x reference)
import functools

import jax
import jax.numpy as jnp
from jax import lax
from jax.experimental import pallas as pl
from jax.experimental.pallas import tpu as pltpu
from jax.experimental.pallas import tpu_sc as plsc

_NUM_WORKERS = 32
_CHUNK_ROWS = 16
_NBUF = 7
_TC_ROWS = 3072
_BLK = 512
_STRIP = 32


def _tc_head(weights, dim):
    nblk = _TC_ROWS // _BLK

    def body(w_hbm, out_hbm, bin_, bout, sin, sout):
        def in_dma(i):
            return pltpu.make_async_copy(
                w_hbm.at[pl.ds(i * _BLK, _BLK + 8)],
                bin_.at[i % 2],
                sin.at[i % 2],
            )

        def out_dma(i):
            return pltpu.make_async_copy(
                bout.at[i % 2],
                out_hbm.at[pl.ds(i * _BLK, _BLK)],
                sout.at[i % 2],
            )

        def shift(slot):
            for r in range(0, _BLK, _STRIP):
                v = bin_[slot, pl.ds(r, _STRIP + 8)]
                bout[slot, pl.ds(r, _STRIP)] = v[1 : _STRIP + 1]

        in_dma(0).start()
        for i in range(nblk):
            if i + 1 < nblk:
                in_dma(i + 1).start()
            in_dma(i).wait()
            if i >= 2:
                out_dma(i - 2).wait()
            shift(i % 2)
            out_dma(i).start()
        for i in range(max(0, nblk - 2), nblk):
            out_dma(i).wait()

    return pl.pallas_call(
        body,
        out_shape=jax.ShapeDtypeStruct((_TC_ROWS, dim), jnp.float32),
        in_specs=[pl.BlockSpec(memory_space=pl.ANY)],
        out_specs=pl.BlockSpec(memory_space=pl.ANY),
        scratch_shapes=[
            pltpu.VMEM((2, _BLK + 8, dim), jnp.float32),
            pltpu.VMEM((2, _BLK, dim), jnp.float32),
            pltpu.SemaphoreType.DMA((2,)),
            pltpu.SemaphoreType.DMA((2,)),
        ],
    )(weights)


def kernel(inputs, weights):
    bsz, seq_len = inputs.shape
    out_rows = 2 * seq_len
    dim = weights.shape[1]
    row_off = (weights.shape[0] // 2 + 1) - seq_len
    rows_per_w = (out_rows - _TC_ROWS) // _NUM_WORKERS
    nchunks = rows_per_w // _CHUNK_ROWS

    mesh = plsc.VectorSubcoreMesh(core_axis_name="c", subcore_axis_name="s")

    @functools.partial(
        pl.kernel,
        mesh=mesh,
        out_type=jax.ShapeDtypeStruct((out_rows, dim), jnp.float32),
        scratch_types=[pltpu.VMEM((_CHUNK_ROWS, dim), jnp.float32)] * _NBUF
        + [pltpu.VMEM((_CHUNK_ROWS,), jnp.int32)] * _NBUF
        + [
            pltpu.SemaphoreType.DMA,
            pltpu.SemaphoreType.DMA,
        ],
    )
    def copy_k(w_hbm, out_hbm, *rest):
        bufs = rest[:_NBUF]
        idxs = rest[_NBUF : 2 * _NBUF]
        sem_in, sem_out = rest[2 * _NBUF :]
        wid = lax.axis_index("s") * 2 + lax.axis_index("c")
        base = _TC_ROWS + wid * rows_per_w

        def in_copy(i):
            b = i % _NBUF
            start = base + row_off + i * _CHUNK_ROWS
            for k in range(_CHUNK_ROWS // 16):
                idxs[b][pl.ds(k * 16, 16)] = start + k * 16 + lax.iota(
                    jnp.int32, 16
                )
            return pltpu.async_copy(w_hbm.at[idxs[b]], bufs[b], sem_in)

        def out_copy(i):
            return pltpu.make_async_copy(
                bufs[i % _NBUF],
                out_hbm.at[pl.ds(base + i * _CHUNK_ROWS, _CHUNK_ROWS)],
                sem_out,
            )

        pending = []
        for j in range(min(_NBUF - 1, nchunks)):
            pending.append(in_copy(j))
        for i in range(nchunks):
            j = i + _NBUF - 1
            if j < nchunks:
                if j >= _NBUF:
                    out_copy(j - _NBUF).wait()
                pending.append(in_copy(j))
            pending.pop(0).wait()
            out_copy(i).start()
        for i in range(max(0, nchunks - _NBUF), nchunks):
            out_copy(i).wait()

    sc_full = copy_k(weights)
    tc_part = _tc_head(weights, dim)
    return lax.dynamic_update_slice(sc_full, tc_part, (0, 0))


# speedup vs baseline: 1.1966x; 1.1966x over previous
import functools

import jax
import jax.numpy as jnp
from jax import lax
from jax.experimental import pallas as pl
from jax.experimental.pallas import tpu as pltpu
from jax.experimental.pallas import tpu_sc as plsc

_NUM_WORKERS = 32
_CHUNK_ROWS = 16
_NBUF = 7


def kernel(inputs, weights):
    bsz, seq_len = inputs.shape
    out_rows = 2 * seq_len
    dim = weights.shape[1]
    row_off = (weights.shape[0] // 2 + 1) - seq_len
    rows_per_w = out_rows // _NUM_WORKERS
    nchunks = rows_per_w // _CHUNK_ROWS

    mesh = plsc.VectorSubcoreMesh(core_axis_name="c", subcore_axis_name="s")

    @functools.partial(
        pl.kernel,
        mesh=mesh,
        out_type=jax.ShapeDtypeStruct((out_rows, dim), jnp.float32),
        scratch_types=[pltpu.VMEM((_CHUNK_ROWS, dim), jnp.float32)] * _NBUF
        + [pltpu.VMEM((_CHUNK_ROWS,), jnp.int32)] * _NBUF
        + [
            pltpu.SemaphoreType.DMA,
            pltpu.SemaphoreType.DMA,
        ],
    )
    def copy_k(w_hbm, out_hbm, *rest):
        bufs = rest[:_NBUF]
        idxs = rest[_NBUF : 2 * _NBUF]
        sem_in, sem_out = rest[2 * _NBUF :]
        wid = lax.axis_index("s") * 2 + lax.axis_index("c")
        base = wid * rows_per_w

        def in_copy(i):
            b = i % _NBUF
            start = base + row_off + i * _CHUNK_ROWS
            for k in range(_CHUNK_ROWS // 16):
                idxs[b][pl.ds(k * 16, 16)] = start + k * 16 + lax.iota(
                    jnp.int32, 16
                )
            return pltpu.async_copy(w_hbm.at[idxs[b]], bufs[b], sem_in)

        def out_copy(i):
            return pltpu.make_async_copy(
                bufs[i % _NBUF],
                out_hbm.at[pl.ds(base + i * _CHUNK_ROWS, _CHUNK_ROWS)],
                sem_out,
            )

        pending = []
        for j in range(min(_NBUF - 1, nchunks)):
            pending.append(in_copy(j))
        for i in range(nchunks):
            j = i + _NBUF - 1
            if j < nchunks:
                if j >= _NBUF:
                    out_copy(j - _NBUF).wait()
                pending.append(in_copy(j))
            pending.pop(0).wait()
            out_copy(i).start()
        for i in range(max(0, nchunks - _NBUF), nchunks):
            out_copy(i).wait()

    return copy_k(weights)
